# baseline (speedup 1.0000x reference)
import functools

import jax
import jax.numpy as jnp
from jax import lax
from jax.experimental import pallas as pl
from jax.experimental.pallas import tpu as pltpu
from jax.experimental.pallas import tpu_sc as plsc

K = 8
BF = jnp.bfloat16
F32 = jnp.float32


def _dense_body(x_ref, wdr_ref, bdr_ref, wa1_ref, ba1_ref, wa2_ref,
                ba2_ref, mid_ref, aa_ref):
  x = x_ref[...].astype(BF)
  mid = jnp.maximum(
      jnp.dot(x, wdr_ref[...], preferred_element_type=F32) + bdr_ref[...],
      0.0)
  t = jnp.tanh(
      jnp.dot(mid.astype(BF), wa1_ref[...], preferred_element_type=F32)
      + ba1_ref[...])
  aa = jnp.dot(t.astype(BF), wa2_ref[...],
               preferred_element_type=F32)[:, 0:1] + ba2_ref[...]
  mid_ref[...] = mid
  aa_ref[...] = aa


def _dense(tfeat, W_dr, b_dr, Wa1, ba1, Wa2, ba2, br):
  n, in_dim = tfeat.shape
  m = W_dr.shape[1]
  h = Wa1.shape[1]
  full = lambda shape: pl.BlockSpec(shape, lambda i: (0, 0))
  wa2p = jnp.pad(Wa2, ((0, 0), (0, 128 - Wa2.shape[1])))
  return pl.pallas_call(
      _dense_body,
      grid=(n // br,),
      in_specs=[
          pl.BlockSpec((br, in_dim), lambda i: (i, 0)),
          full((in_dim, m)),
          full((1, m)),
          full((m, h)),
          full((1, h)),
          full((h, 128)),
          full((1, 1)),
      ],
      out_specs=[
          pl.BlockSpec((br, m), lambda i: (i, 0)),
          pl.BlockSpec((br, 1), lambda i: (i, 0)),
      ],
      out_shape=[
          jax.ShapeDtypeStruct((n, m), F32),
          jax.ShapeDtypeStruct((n, 1), F32),
      ],
  )(tfeat, W_dr.astype(BF), b_dr.reshape(1, m), Wa1.astype(BF),
    ba1.reshape(1, h), wa2p.astype(BF), ba2.reshape(1, 1))


def _sc_gather(idx_flat, aa, mid, nw, nc, batch):
  b = idx_flat.shape[0]
  m = mid.shape[1]
  npw = b // nw
  nb = npw // batch
  mesh = plsc.VectorSubcoreMesh(core_axis_name="c", subcore_axis_name="s")

  @functools.partial(
      pl.kernel,
      out_type=[
          jax.ShapeDtypeStruct((b,), F32),
          jax.ShapeDtypeStruct((b, m), F32),
      ],
      mesh=mesh,
      scratch_types=[
          pltpu.VMEM((npw,), jnp.int32),
          pltpu.VMEM((npw,), F32),
          pltpu.VMEM((batch,), jnp.int32),
          pltpu.VMEM((batch, m), F32),
          pltpu.SemaphoreType.DMA,
      ],
  )
  def body(idx_hbm, aa_hbm, mid_hbm, oa_hbm, om_hbm,
           idx_v, aa_v, idx_b, rows_v, sem):
    wid = lax.axis_index("s") * nc + lax.axis_index("c")
    base = wid * npw
    pltpu.sync_copy(idx_hbm.at[pl.ds(base, npw)], idx_v)
    c0 = pltpu.make_async_copy(aa_hbm.at[idx_v], aa_v, sem)
    c0.start()
    c0.wait()
    pltpu.sync_copy(aa_v, oa_hbm.at[pl.ds(base, npw)])

    def step(i, _):
      off = base + i * batch
      pltpu.sync_copy(idx_hbm.at[pl.ds(off, batch)], idx_b)
      c = pltpu.make_async_copy(mid_hbm.at[idx_b], rows_v, sem)
      c.start()
      c.wait()
      pltpu.sync_copy(rows_v, om_hbm.at[pl.ds(off, batch)])
      return 0

    lax.fori_loop(0, nb, step, 0)

  return body(idx_flat, aa, mid)


def _make_chunkmm_body(chunk, cp, st, m):
  r = cp // 128
  rt = st // 128

  def body(aa_ref, aasub_ref, gmid_ref, wc_ref, key_ref, tatt_ref):
    t = pl.program_id(1)
    aa = aa_ref[0]
    pos = (lax.broadcasted_iota(jnp.int32, (r, 128), 0) * 128
           + lax.broadcasted_iota(jnp.int32, (r, 128), 1))
    valid = pos < chunk
    mx = jnp.max(jnp.where(valid, aa, jnp.float32(-jnp.inf)))
    z = jnp.sum(jnp.where(valid, jnp.exp(aa - mx), 0.0))
    aas = aasub_ref[0]
    rowpos = t * st + lax.broadcasted_iota(jnp.int32, (st, 1), 0)
    w = jnp.where(rowpos < chunk, jnp.exp(aas - mx) / z, 0.0)
    af = gmid_ref[0] * w
    logits = jnp.dot(af.astype(BF), wc_ref[...], preferred_element_type=F32)
    l0 = logits[:, 0:1]
    l1 = logits[:, 1:2]
    m2 = jnp.maximum(l0, l1)
    e0 = jnp.exp(l0 - m2)
    e1 = jnp.exp(l1 - m2)
    key_ref[0] = e1 / (e0 + e1)
    part = jnp.sum(af, axis=0, keepdims=True)

    @pl.when(t == 0)
    def _():
      tatt_ref[0] = part

    @pl.when(t != 0)
    def _():
      tatt_ref[0] = tatt_ref[0] + part

  return body


def _chunkmm(aa_g, gmid, Wc, chunk, cp, st):
  g = aa_g.size // cp
  r = cp // 128
  m = gmid.shape[1]
  nt = cp // st
  body = _make_chunkmm_body(chunk, cp, st, m)
  keys, tatt = pl.pallas_call(
      body,
      grid=(g, nt),
      in_specs=[
          pl.BlockSpec((1, r, 128), lambda i, t: (i, 0, 0)),
          pl.BlockSpec((1, st, 1), lambda i, t: (i, t, 0)),
          pl.BlockSpec((1, st, m), lambda i, t: (i, t, 0)),
          pl.BlockSpec((m, 2), lambda i, t: (0, 0)),
      ],
      out_specs=[
          pl.BlockSpec((1, st, 1), lambda i, t: (i, t, 0)),
          pl.BlockSpec((1, 1, m), lambda i, t: (i, 0, 0)),
      ],
      out_shape=[
          jax.ShapeDtypeStruct((g, cp, 1), F32),
          jax.ShapeDtypeStruct((g, 1, m), F32),
      ],
  )(aa_g.reshape(g, r, 128), aa_g.reshape(g, cp, 1),
    gmid.reshape(g, cp, m), Wc.astype(BF))
  return keys, tatt


def _make_select_body(chunk, cp, m):
  r = cp // 128

  def body(key_ref, tatt_ref, gmid_ref, wc_ref, bc_ref, pred_ref, feat_ref,
           rows_v, sem):
    gi = pl.program_id(0)
    key = key_ref[0]
    pos = (lax.broadcasted_iota(jnp.int32, (r, 128), 0) * 128
           + lax.broadcasted_iota(jnp.int32, (r, 128), 1))
    valid = pos < chunk
    neg = jnp.float32(-jnp.inf)
    posf = jnp.float32(jnp.inf)
    km = jnp.where(valid, key, neg)
    kn = jnp.where(valid, key, posf)
    sel = []
    for _ in range(K):
      mv = jnp.max(km)
      j = jnp.min(jnp.where(km == mv, pos, cp))
      sel.append(j)
      km = jnp.where(pos == j, neg, km)
    mins = []
    for _ in range(K):
      mv = jnp.min(kn)
      j = jnp.max(jnp.where(kn == mv, pos, -1))
      mins.append(j)
      kn = jnp.where(pos == j, posf, kn)
    order = sel + mins[::-1]
    copies = []
    for t, j in enumerate(order):
      row = gi * cp + j
      c = pltpu.make_async_copy(
          gmid_ref.at[pl.ds(row, 1)], rows_v.at[pl.ds(t, 1)], sem)
      c.start()
      copies.append(c)
    for c in copies:
      c.wait()
    feat_ref[0] = rows_v[...]
    pv = jnp.dot(tatt_ref[0].astype(BF), wc_ref[...],
                 preferred_element_type=F32)
    pred_ref[0] = pv + bc_ref[...]

  return body


def _select(keys, tatt, gmid_flat, Wc, bc, chunk, cp):
  g = keys.shape[0]
  r = cp // 128
  m = gmid_flat.shape[1]
  body = _make_select_body(chunk, cp, m)
  preds, feats = pl.pallas_call(
      body,
      grid=(g,),
      in_specs=[
          pl.BlockSpec((1, r, 128), lambda i: (i, 0, 0)),
          pl.BlockSpec((1, 1, m), lambda i: (i, 0, 0)),
          pl.BlockSpec(memory_space=pl.ANY),
          pl.BlockSpec((m, 2), lambda i: (0, 0)),
          pl.BlockSpec((1, 2), lambda i: (0, 0)),
      ],
      out_specs=[
          pl.BlockSpec((1, 1, 2), lambda i: (i, 0, 0)),
          pl.BlockSpec((1, 2 * K, m), lambda i: (i, 0, 0)),
      ],
      out_shape=[
          jax.ShapeDtypeStruct((g, 1, 2), F32),
          jax.ShapeDtypeStruct((g, 2 * K, m), F32),
      ],
      scratch_shapes=[
          pltpu.VMEM((2 * K, m), F32),
          pltpu.SemaphoreType.DMA,
      ],
  )(keys.reshape(g, r, 128), tatt, gmid_flat, Wc.astype(BF),
    bc.reshape(1, 2))
  return preds, feats


def kernel(tfeat_tensor, index_chunk_list, W_dr, b_dr, Wa1, ba1, Wa2, ba2,
           Wc, bc):
  n, in_dim = tfeat_tensor.shape
  g, chunk = index_chunk_list.shape
  m = W_dr.shape[1]

  br = 1
  for cand in range(min(n, 1024), 7, -1):
    if n % cand == 0 and cand % 8 == 0:
      br = cand
      break
  if br == 1:
    br = n

  mid, aa = _dense(tfeat_tensor, W_dr, b_dr, Wa1, ba1, Wa2, ba2, br)
  aa = aa.reshape(n)

  info = plsc.get_sparse_core_info()
  nc, ns = info.num_cores, info.num_subcores
  nw = nc * ns

  cp = pl.cdiv(chunk, 128) * 128
  while (g * cp) % (8 * nw) != 0 or ((g * cp) // nw) % 8 != 0:
    cp += 128
  idx32 = index_chunk_list.astype(jnp.int32)
  idx_pad = jnp.pad(idx32, ((0, 0), (0, cp - chunk)))

  npw = (g * cp) // nw
  batch = 160 if npw % 160 == 0 else 8
  aa_g, gmid = _sc_gather(idx_pad.reshape(g * cp), aa, mid, nw, nc, batch)

  st = cp
  for cand in range(1280, 127, -128):
    if cp % cand == 0:
      st = cand
      break

  keys, tatt = _chunkmm(aa_g, gmid, Wc, chunk, cp, st)
  preds, feats = _select(keys, tatt, gmid, Wc, bc, chunk, cp)
  return preds, feats
